# baseline (speedup 1.0000x reference)
import functools

import jax
import jax.numpy as jnp
import numpy as np
from jax import lax
from jax.experimental import pallas as pl

_B, _C, _L = 32, 256, 128
_THRES = 0.5
_NSAMP = 30
_NTOP = 30
_NSEL = 32
_SCALE = 1.0 / np.sqrt(_L)
_BPS = 2

_mmd = functools.partial(lax.dot_general, precision=lax.Precision.DEFAULT)
_mmh = functools.partial(lax.dot_general, precision=lax.Precision.HIGHEST)


def _dot(a, b):
    return _mmd(a, b, (((1,), (0,)), ((), ())))


def _dot_t(a, b):
    return _mmd(a, b, (((1,), (1,)), ((), ())))


def _dot_lt(a, b):
    return _mmh(a, b, (((0,), (0,)), ((), ())))


def _bf16(a):
    return a.astype(jnp.bfloat16).astype(jnp.float32)


def _safe_recip(r):
    rinv = 1.0 / r
    return jnp.where(jnp.abs(rinv) == jnp.inf, 0.0, rinv)


def _body(x_ref, wqkv_ref, bqkv_ref, w1_ref, b1_ref, w2_ref, b2_ref,
          gamma_ref, idx_ref, o_ref):
    w1 = w1_ref[...]
    b1 = b1_ref[...]
    w2 = w2_ref[...]
    b2 = b2_ref[...]

    ii = lax.broadcasted_iota(jnp.int32, (_C, _C), 0)
    jj = lax.broadcasted_iota(jnp.int32, (_C, _C), 1)
    eye = (ii == jj).astype(jnp.float32)
    tri = (jj <= ii).astype(jnp.float32)
    lowstrict = (jj < ii).astype(jnp.float32)
    jot2 = lax.broadcasted_iota(jnp.int32, (_L, 2 * _L), 0)
    riota = lax.broadcasted_iota(jnp.int32, (_NSEL, _C), 0).astype(jnp.float32)
    ciota = lax.broadcasted_iota(jnp.int32, (_C, _NSEL), 1).astype(jnp.float32)

    for j in range(_BPS):
        xb = x_ref[j]
        t_all = _dot(wqkv_ref[...], xb) + bqkv_ref[...]

        nrm = jnp.sqrt(jnp.sum(t_all * t_all, axis=1, keepdims=True))
        tn_all = t_all / jnp.maximum(nrm, 1e-8)
        feats_all = t_all * _safe_recip(jnp.sum(t_all, axis=1, keepdims=True))
        g1_all = _dot(feats_all, w1)

        def gcn(c):
            tn = tn_all[c * _C:(c + 1) * _C]
            sim = _dot_t(tn, tn)
            adj = (sim > _THRES).astype(jnp.float32) + eye
            adjn = adj * _safe_recip(jnp.sum(adj, axis=1, keepdims=True))
            h = _dot(adjn, g1_all[c * _C:(c + 1) * _C]) + b1
            h = jnp.maximum(h, 0.0)
            return _dot(adjn, _dot(h, w2)) + b2

        q = gcn(0)
        k = gcn(1)
        v = gcn(2)

        runmax = jnp.full((_C, 1), -jnp.inf, jnp.float32)
        runsum = jnp.zeros((_C, 1), jnp.float32)
        qb = _bf16(q)
        qbb = jnp.concatenate([qb, qb], axis=1)
        for p in range(_NSAMP // 2):
            onehot2 = (jot2 == idx_ref[p:p + 1, :]).astype(jnp.float32)
            ksb2 = _dot(k, onehot2)
            prod = qbb * ksb2
            qk_a = jnp.sum(prod[:, :_L], axis=1, keepdims=True)
            qk_b = jnp.sum(prod[:, _L:], axis=1, keepdims=True)
            runmax = jnp.maximum(jnp.maximum(runmax, qk_a), qk_b)
            runsum = (runsum + qk_a) + qk_b
        m_col = runmax - runsum * (1.0 / _L)

        m_row = _dot_lt(m_col, eye)
        gt = (m_row > m_col).astype(jnp.float32)
        eqlow = (m_row == m_col).astype(jnp.float32) * lowstrict
        rank = jnp.sum(gt + eqlow, axis=1, keepdims=True)
        sel = rank < float(_NTOP)
        rank_row = _dot_lt(rank, eye)
        gather32 = (riota == rank_row).astype(jnp.float32)
        scatter32 = (rank == ciota).astype(jnp.float32)

        q32 = _dot(gather32, q)
        scores = _dot_t(q32, k) * _SCALE
        smax = jnp.max(scores, axis=1, keepdims=True)
        e = jnp.exp(scores - smax)
        attn = e / jnp.sum(e, axis=1, keepdims=True)
        upd32 = _dot(attn, v)
        upd = _dot(scatter32, upd32)

        v_hi = _bf16(v)
        ctx = _dot(tri, v_hi) + _dot(tri, v - v_hi)
        ctx = jnp.where(sel, upd, ctx)

        o_ref[j] = gamma_ref[...] * ctx + xb


def kernel(x, Wq, bq, Wk, bk, Wv, bv, W1, b1, W2, b2, gamma):
    idx = jax.random.randint(jax.random.key(42), (_L, _NSAMP), 0, _L)
    idx_pairs = idx.astype(jnp.int32).T.reshape(_NSAMP // 2, 2 * _L)
    idx_pad = jnp.zeros((16, 2 * _L), jnp.int32).at[:_NSAMP // 2].set(idx_pairs)

    wqkv = jnp.concatenate([Wq, Wk, Wv], axis=0)
    bqkv = jnp.concatenate([bq, bk, bv]).reshape(3 * _C, 1)

    full = lambda shape: pl.BlockSpec(shape, lambda b: (0,) * len(shape))
    out = pl.pallas_call(
        _body,
        grid=(_B // _BPS,),
        in_specs=[
            pl.BlockSpec((_BPS, _C, _L), lambda b: (b, 0, 0)),
            full((3 * _C, _C)), full((3 * _C, 1)),
            full((_L, 8)), full((1, 8)),
            full((8, _L)), full((1, _L)),
            full((1, 1)), full((16, 2 * _L)),
        ],
        out_specs=pl.BlockSpec((_BPS, _C, _L), lambda b: (b, 0, 0)),
        out_shape=jax.ShapeDtypeStruct((_B, _C, _L), jnp.float32),
    )(x, wqkv, bqkv, W1, b1.reshape(1, 8), W2, b2.reshape(1, _L),
      gamma.reshape(1, 1), idx_pad)
    return out


# speedup vs baseline: 1.2262x; 1.2262x over previous
import functools

import jax
import jax.numpy as jnp
import numpy as np
from jax import lax
from jax.experimental import pallas as pl

_B, _C, _L = 32, 256, 128
_THRES = 0.5
_NSAMP = 30
_NTOP = 30
_NSEL = 32
_SCALE = 1.0 / np.sqrt(_L)
_BPS = 4

_mmd = functools.partial(lax.dot_general, precision=lax.Precision.DEFAULT)
_mmh = functools.partial(lax.dot_general, precision=lax.Precision.HIGHEST)


def _dot(a, b):
    return _mmd(a, b, (((1,), (0,)), ((), ())))


def _dot_t(a, b):
    return _mmd(a, b, (((1,), (1,)), ((), ())))


def _dot_lt(a, b):
    return _mmh(a, b, (((0,), (0,)), ((), ())))


def _bf16(a):
    return a.astype(jnp.bfloat16).astype(jnp.float32)


def _safe_recip(r):
    rinv = 1.0 / r
    return jnp.where(jnp.abs(rinv) == jnp.inf, 0.0, rinv)


def _body(x_ref, wqkv_ref, bqkv_ref, w1_ref, b1_ref, w2_ref, b2_ref,
          gamma_ref, idx_ref, o_ref):
    w1 = w1_ref[...]
    b1 = b1_ref[...]
    w2 = w2_ref[...]
    b2 = b2_ref[...]

    ii = lax.broadcasted_iota(jnp.int32, (_C, _C), 0)
    jj = lax.broadcasted_iota(jnp.int32, (_C, _C), 1)
    eye = (ii == jj).astype(jnp.float32)
    tri = (jj <= ii).astype(jnp.float32)
    lowstrict = (jj < ii).astype(jnp.float32)
    jot2 = lax.broadcasted_iota(jnp.int32, (_L, 2 * _L), 0)

    for j in range(_BPS):
        xb = x_ref[j]
        t_all = _dot(wqkv_ref[...], xb) + bqkv_ref[...]

        def cos_gcn(t):
            nrm = jnp.sqrt(jnp.sum(t * t, axis=1, keepdims=True))
            tn = t / jnp.maximum(nrm, 1e-8)
            sim = _dot_t(tn, tn)
            adj = (sim > _THRES).astype(jnp.float32) + eye
            adjn = adj * _safe_recip(jnp.sum(adj, axis=1, keepdims=True))
            feats = t * _safe_recip(jnp.sum(t, axis=1, keepdims=True))
            h = _dot(adjn, _dot(feats, w1)) + b1
            h = jnp.maximum(h, 0.0)
            return _dot(adjn, _dot(h, w2)) + b2

        q = cos_gcn(t_all[0 * _C:1 * _C])
        k = cos_gcn(t_all[1 * _C:2 * _C])
        v = cos_gcn(t_all[2 * _C:3 * _C])

        runmax = jnp.full((_C, 1), -jnp.inf, jnp.float32)
        runsum = jnp.zeros((_C, 1), jnp.float32)
        qb = _bf16(q)
        qbb = jnp.concatenate([qb, qb], axis=1)
        for p in range(_NSAMP // 2):
            onehot2 = (jot2 == idx_ref[p:p + 1, :]).astype(jnp.float32)
            ksb2 = _dot(k, onehot2)
            prod = qbb * ksb2
            qk_a = jnp.sum(prod[:, :_L], axis=1, keepdims=True)
            qk_b = jnp.sum(prod[:, _L:], axis=1, keepdims=True)
            runmax = jnp.maximum(jnp.maximum(runmax, qk_a), qk_b)
            runsum = (runsum + qk_a) + qk_b
        m_col = runmax - runsum * (1.0 / _L)

        m_row = _dot_lt(m_col, eye)
        gt = (m_row > m_col).astype(jnp.float32)
        eqlow = (m_row == m_col).astype(jnp.float32) * lowstrict
        rank = jnp.sum(gt + eqlow, axis=1, keepdims=True)
        sel = rank < float(_NTOP)

        scores = _dot_t(q, k) * _SCALE
        smax = jnp.max(scores, axis=1, keepdims=True)
        e = jnp.exp(scores - smax)
        attn = e / jnp.sum(e, axis=1, keepdims=True)
        upd = _dot(attn, v)

        v_hi = _bf16(v)
        ctx = _dot(tri, v_hi) + _dot(tri, v - v_hi)
        ctx = jnp.where(sel, upd, ctx)

        o_ref[j] = gamma_ref[...] * ctx + xb


def kernel(x, Wq, bq, Wk, bk, Wv, bv, W1, b1, W2, b2, gamma):
    idx = jax.random.randint(jax.random.key(42), (_L, _NSAMP), 0, _L)
    idx_pairs = idx.astype(jnp.int32).T.reshape(_NSAMP // 2, 2 * _L)
    idx_pad = jnp.zeros((16, 2 * _L), jnp.int32).at[:_NSAMP // 2].set(idx_pairs)

    wqkv = jnp.concatenate([Wq, Wk, Wv], axis=0)
    bqkv = jnp.concatenate([bq, bk, bv]).reshape(3 * _C, 1)

    full = lambda shape: pl.BlockSpec(shape, lambda b: (0,) * len(shape))
    out = pl.pallas_call(
        _body,
        grid=(_B // _BPS,),
        in_specs=[
            pl.BlockSpec((_BPS, _C, _L), lambda b: (b, 0, 0)),
            full((3 * _C, _C)), full((3 * _C, 1)),
            full((_L, 8)), full((1, 8)),
            full((8, _L)), full((1, _L)),
            full((1, 1)), full((16, 2 * _L)),
        ],
        out_specs=pl.BlockSpec((_BPS, _C, _L), lambda b: (b, 0, 0)),
        out_shape=jax.ShapeDtypeStruct((_B, _C, _L), jnp.float32),
    )(x, wqkv, bqkv, W1, b1.reshape(1, 8), W2, b2.reshape(1, _L),
      gamma.reshape(1, 1), idx_pad)
    return out
